# baseline (speedup 1.0000x reference)
import functools

import jax
import jax.numpy as jnp
from jax import lax
from jax.experimental import pallas as pl
from jax.experimental.pallas import tpu as pltpu
from jax.experimental.pallas import tpu_sc as plsc

N_NODES = 10000
N_EDGES = 160000
D_FEAT = 128
D_EDGE = 16
EMB = 64
HEADS = 2
N_GRAPHS = 32
HC = HEADS * EMB

NC = 2
NS = 16
E_PAD = 163840
CH = 32
N_CHROWS = E_PAD // CH
CPT = N_CHROWS // NS
IGRP = 16
N_IGRP = CPT // IGRP
N_ACC = 10240
ROWS_PER_TILE = N_ACC // NS

EB = 1280
N_EBLK = E_PAD // EB
NB = 1000
N_NBLK = N_NODES // NB
PB = 400
N_PBLK = N_NODES // PB

_mesh = plsc.VectorSubcoreMesh(
    core_axis_name="c", subcore_axis_name="s", num_cores=NC, num_subcores=NS)

_GDN = lax.GatherDimensionNumbers(
    offset_dims=(), collapsed_slice_dims=(0,), start_index_map=(0,))


def _lanesum(v):
  for k in (1, 2, 4, 8):
    perm = lax.iota(jnp.int32, 16) ^ k
    v = v + lax.gather(v, perm[:, None], _GDN, (1,),
                       mode=lax.GatherScatterMode.PROMISE_IN_BOUNDS)
  return v


def _edge_sc_body(t_hbm, q_hbm, e_hbm, src_hbm, dstq_hbm, dst_hbm, zrows_hbm,
                  part_hbm, acc, idx_s, idx_dq, idx_d,
                  tbufA, qbufA, ebufA, obufA, tbufB, qbufB, ebufB, obufB,
                  semA_t, semA_q, semA_e, semB_t, semB_q, semB_e):
  cid = lax.axis_index("c")
  sid = lax.axis_index("s")
  pltpu.sync_copy(zrows_hbm, acc.at[pl.ds(sid * ROWS_PER_TILE, ROWS_PER_TILE)])

  def zrow(r, _):
    z16 = jnp.zeros((16,), jnp.float32)
    for ob in (obufA, obufB):
      ob[r, 80:96] = z16
      ob[r, 96:112] = z16
      ob[r, 112:128] = z16
    return 0

  lax.fori_loop(0, CH, zrow, 0)
  plsc.subcore_barrier()

  row0 = sid * CPT

  def start(bufs, sems, grow, c):
    tb, qb, eb = bufs
    st, sq, se = sems
    ct = pltpu.make_async_copy(t_hbm.at[idx_s.at[c]], tb, st)
    cq = pltpu.make_async_copy(q_hbm.at[idx_dq.at[c]], qb, sq)
    ce = pltpu.make_async_copy(
        e_hbm.at[cid, pl.ds((grow + c) * CH, CH)], eb, se)
    ct.start()
    cq.start()
    ce.start()

  def wait(bufs, sems):
    tb, qb, eb = bufs
    st, sq, se = sems
    pltpu.make_async_copy(t_hbm.at[idx_s.at[0]], tb, st).wait()
    pltpu.make_async_copy(q_hbm.at[idx_dq.at[0]], qb, sq).wait()
    pltpu.make_async_copy(e_hbm.at[cid, pl.ds(row0 * CH, CH)], eb, se).wait()

  def compute_scatter(bufs, obuf, grow, c):
    tbuf, qbuf, ebuf = bufs
    base = (grow + c) * CH

    @plsc.parallel_loop(0, CH, unroll=4)
    def edge(r):
      e0 = ebuf[r, 0:16]
      e1 = ebuf[r, 16:32]
      e2 = ebuf[r, 32:48]
      e3 = ebuf[r, 48:64]
      s16 = (qbuf[r, 0:16] * (tbuf[r, 0:16] + e0)
             + qbuf[r, 16:32] * (tbuf[r, 16:32] + e1)
             + qbuf[r, 32:48] * (tbuf[r, 32:48] + e2)
             + qbuf[r, 48:64] * (tbuf[r, 48:64] + e3))
      live = (base + r < N_EDGES).astype(jnp.float32)
      exv = jnp.exp(_lanesum(s16) * 0.125) * live
      obuf[r, 0:16] = (tbuf[r, 64:80] + e0) * exv
      obuf[r, 16:32] = (tbuf[r, 80:96] + e1) * exv
      obuf[r, 32:48] = (tbuf[r, 96:112] + e2) * exv
      obuf[r, 48:64] = (tbuf[r, 112:128] + e3) * exv
      obuf[r, 64:80] = exv

    pltpu.sync_copy(obuf, acc.at[idx_d.at[c]], add=True)

  A = (tbufA, qbufA, ebufA)
  B = (tbufB, qbufB, ebufB)
  SA = (semA_t, semA_q, semA_e)
  SB = (semB_t, semB_q, semB_e)

  def group(g, _):
    grow = row0 + g * IGRP
    pltpu.sync_copy(src_hbm.at[cid, pl.ds(grow, IGRP)], idx_s)
    pltpu.sync_copy(dstq_hbm.at[cid, pl.ds(grow, IGRP)], idx_dq)
    pltpu.sync_copy(dst_hbm.at[pl.ds(grow, IGRP)], idx_d)
    start(A, SA, grow, 0)

    def pair(i, _):
      c0 = 2 * i
      c1 = c0 + 1
      c2 = jnp.where(i < IGRP // 2 - 1, c0 + 2, 0)
      wait(A, SA)
      start(B, SB, grow, c1)
      compute_scatter(A, obufA, grow, c0)
      wait(B, SB)
      start(A, SA, grow, c2)
      compute_scatter(B, obufB, grow, c1)
      return 0

    lax.fori_loop(0, IGRP // 2, pair, 0)
    wait(A, SA)
    return 0

  lax.fori_loop(0, N_IGRP, group, 0)
  plsc.subcore_barrier()
  pltpu.sync_copy(acc.at[pl.ds(sid * ROWS_PER_TILE, ROWS_PER_TILE)],
                  part_hbm.at[cid, pl.ds(sid * ROWS_PER_TILE, ROWS_PER_TILE)])


_edge_sc = pl.kernel(
    _edge_sc_body,
    out_type=jax.ShapeDtypeStruct((NC, N_ACC, HC), jnp.float32),
    mesh=_mesh,
    scratch_types=[pltpu.VMEM_SHARED((N_ACC, HC), jnp.float32),
                   pltpu.VMEM((IGRP, CH), jnp.int32),
                   pltpu.VMEM((IGRP, CH), jnp.int32),
                   pltpu.VMEM((IGRP, CH), jnp.int32),
                   pltpu.VMEM((CH, HC), jnp.float32),
                   pltpu.VMEM((CH, HC), jnp.float32),
                   pltpu.VMEM((CH, EMB), jnp.float32),
                   pltpu.VMEM((CH, HC), jnp.float32),
                   pltpu.VMEM((CH, HC), jnp.float32),
                   pltpu.VMEM((CH, HC), jnp.float32),
                   pltpu.VMEM((CH, EMB), jnp.float32),
                   pltpu.VMEM((CH, HC), jnp.float32),
                   pltpu.SemaphoreType.DMA,
                   pltpu.SemaphoreType.DMA,
                   pltpu.SemaphoreType.DMA,
                   pltpu.SemaphoreType.DMA,
                   pltpu.SemaphoreType.DMA,
                   pltpu.SemaphoreType.DMA],
)


def _mm_body(norm, x_ref, w_ref, b_ref, st_ref, g_ref, bb_ref,
             t_ref, q_ref, sk_ref):
  x = x_ref[...]
  if norm:
    inv_n = 1.0 / N_NODES
    mu = st_ref[0:1, :] * inv_n
    var = st_ref[1:2, :] * inv_n - mu * mu
    s = g_ref[...] * lax.rsqrt(var + 1e-5)
    t = bb_ref[...] - mu * s
    x = x * s + t
  y = jnp.dot(x, w_ref[...], preferred_element_type=jnp.float32) + b_ref[...]
  t_ref[...] = jnp.stack([y[:, :HC], y[:, HC:2 * HC]], axis=0)
  z = jnp.zeros((y.shape[0], EMB), jnp.float32)
  q_ref[...] = jnp.stack(
      [jnp.concatenate([y[:, 2 * HC:2 * HC + EMB], z], axis=1),
       jnp.concatenate([y[:, 2 * HC + EMB:3 * HC], z], axis=1)], axis=0)
  sk_ref[...] = y[:, 3 * HC:]


def _run_matmul(x, wall, ball, stats, g, b, norm):
  d = x.shape[1]
  body = functools.partial(_mm_body, norm)
  return pl.pallas_call(
      body,
      grid=(N_NBLK,),
      in_specs=[
          pl.BlockSpec((NB, d), lambda i: (i, 0)),
          pl.BlockSpec((d, 4 * HC), lambda i: (0, 0)),
          pl.BlockSpec((1, 4 * HC), lambda i: (0, 0)),
          pl.BlockSpec((2, d), lambda i: (0, 0)),
          pl.BlockSpec((1, d), lambda i: (0, 0)),
          pl.BlockSpec((1, d), lambda i: (0, 0)),
      ],
      out_specs=[
          pl.BlockSpec((NC, NB, HC), lambda i: (0, i, 0)),
          pl.BlockSpec((NC, NB, HC), lambda i: (0, i, 0)),
          pl.BlockSpec((NB, HC), lambda i: (i, 0)),
      ],
      out_shape=[
          jax.ShapeDtypeStruct((NC, N_NODES, HC), jnp.float32),
          jax.ShapeDtypeStruct((NC, N_NODES, HC), jnp.float32),
          jax.ShapeDtypeStruct((N_NODES, HC), jnp.float32),
      ],
  )(x, wall, ball, stats, g, b)


def _eproj_body(ea_ref, we_ref, be_ref, eh_ref):
  e = jnp.dot(ea_ref[...], we_ref[...],
              preferred_element_type=jnp.float32) + be_ref[...]
  eh_ref[...] = jnp.stack([e[:, :EMB], e[:, EMB:]], axis=0)


def _run_eproj(ea_pad, we, be):
  return pl.pallas_call(
      _eproj_body,
      grid=(N_EBLK,),
      in_specs=[
          pl.BlockSpec((EB, D_EDGE), lambda i: (i, 0)),
          pl.BlockSpec((D_EDGE, HC), lambda i: (0, 0)),
          pl.BlockSpec((1, HC), lambda i: (0, 0)),
      ],
      out_specs=pl.BlockSpec((NC, EB, EMB), lambda i: (0, i, 0)),
      out_shape=jax.ShapeDtypeStruct((NC, E_PAD, EMB), jnp.float32),
  )(ea_pad, we, be)


def _combine_body(part_ref, sk_ref, wa_ref, wb_ref,
                  wt_ref, bt_ref, h2_ref, st_ref, sacc):
  pid = pl.program_id(0)
  p = part_ref[...]
  num = jnp.concatenate([p[0][:, :EMB], p[1][:, :EMB]], axis=1)
  d0 = p[0][:, EMB:EMB + 16]
  d1 = p[1][:, EMB:EMB + 16]
  den = jnp.concatenate([d0, d0, d0, d0, d1, d1, d1, d1], axis=1)
  out = num / (den + 1e-16)
  xr = sk_ref[...]
  bpre = (jnp.sum(out * wa_ref[...], axis=1, keepdims=True)
          + jnp.sum(xr * wb_ref[...], axis=1, keepdims=True))
  beta = jax.nn.sigmoid(bpre)
  h = beta * xr + (1.0 - beta) * out
  y = jnp.dot(h, wt_ref[...], preferred_element_type=jnp.float32) + bt_ref[...]
  h2 = jnp.where(y > 0, y, 0.01 * y)
  h2_ref[...] = h2

  @pl.when(pid == 0)
  def _():
    sacc[...] = jnp.zeros_like(sacc)

  sacc[0:1, :] += jnp.sum(h2, axis=0, keepdims=True)
  sacc[1:2, :] += jnp.sum(h2 * h2, axis=0, keepdims=True)

  @pl.when(pid == N_NBLK - 1)
  def _():
    st_ref[...] = sacc[...]


def _run_combine(part, sk, wa, wb, wt, bt):
  return pl.pallas_call(
      _combine_body,
      grid=(N_NBLK,),
      in_specs=[
          pl.BlockSpec((NC, NB, HC), lambda i: (0, i, 0)),
          pl.BlockSpec((NB, HC), lambda i: (i, 0)),
          pl.BlockSpec((1, HC), lambda i: (0, 0)),
          pl.BlockSpec((1, HC), lambda i: (0, 0)),
          pl.BlockSpec((HC, EMB), lambda i: (0, 0)),
          pl.BlockSpec((1, EMB), lambda i: (0, 0)),
      ],
      out_specs=[
          pl.BlockSpec((NB, EMB), lambda i: (i, 0)),
          pl.BlockSpec((2, EMB), lambda i: (0, 0)),
      ],
      out_shape=[
          jax.ShapeDtypeStruct((N_NODES, EMB), jnp.float32),
          jax.ShapeDtypeStruct((2, EMB), jnp.float32),
      ],
      scratch_shapes=[pltpu.VMEM((2, EMB), jnp.float32)],
  )(part, sk, wa, wb, wt, bt)


def _pool_body(h2_ref, st_ref, g_ref, bb_ref, bi_ref, prev_ref, rep_ref,
               mxacc, smacc, ctacc):
  pid = pl.program_id(0)
  inv_n = 1.0 / N_NODES
  mu = st_ref[0:1, :] * inv_n
  var = st_ref[1:2, :] * inv_n - mu * mu
  s = g_ref[...] * lax.rsqrt(var + 1e-5)
  t = bb_ref[...] - mu * s
  xn = h2_ref[...] * s + t
  bif = bi_ref[...]
  neg = jnp.float32(-jnp.inf)

  @pl.when(pid == 0)
  def _():
    mxacc[...] = jnp.full_like(mxacc, neg)
    smacc[...] = jnp.zeros_like(smacc)
    ctacc[...] = jnp.zeros_like(ctacc)

  for g in range(N_GRAPHS):
    mk = bif == float(g)
    sel = jnp.where(mk, xn, neg)
    mxacc[g:g + 1, :] = jnp.maximum(
        mxacc[g:g + 1, :], jnp.max(sel, axis=0, keepdims=True))
    smacc[g:g + 1, :] += jnp.sum(jnp.where(mk, xn, 0.0), axis=0, keepdims=True)
    ctacc[g:g + 1, :] += jnp.sum(mk.astype(jnp.float32), axis=0, keepdims=True)

  @pl.when(pid == N_PBLK - 1)
  def _():
    mean = smacc[...] / jnp.maximum(ctacc[...], 1.0)
    mx = mxacc[...]
    mx = jnp.where(mx == neg, 0.0, mx)
    rep_ref[...] = jnp.concatenate([mx, mean], axis=1) + prev_ref[...]


def _run_pool(h2, stats, g, b, bif, prev):
  return pl.pallas_call(
      _pool_body,
      grid=(N_PBLK,),
      in_specs=[
          pl.BlockSpec((PB, EMB), lambda i: (i, 0)),
          pl.BlockSpec((2, EMB), lambda i: (0, 0)),
          pl.BlockSpec((1, EMB), lambda i: (0, 0)),
          pl.BlockSpec((1, EMB), lambda i: (0, 0)),
          pl.BlockSpec((PB, 1), lambda i: (i, 0)),
          pl.BlockSpec((N_GRAPHS, 2 * EMB), lambda i: (0, 0)),
      ],
      out_specs=pl.BlockSpec((N_GRAPHS, 2 * EMB), lambda i: (0, 0)),
      out_shape=jax.ShapeDtypeStruct((N_GRAPHS, 2 * EMB), jnp.float32),
      scratch_shapes=[pltpu.VMEM((N_GRAPHS, EMB), jnp.float32),
                      pltpu.VMEM((N_GRAPHS, EMB), jnp.float32),
                      pltpu.VMEM((N_GRAPHS, 1), jnp.float32)],
  )(h2, stats, g, b, bif, prev)


def _prep_block(p):
  cv = p['conv']
  wall = jnp.concatenate([cv['Wk'][:, :EMB], cv['Wv'][:, :EMB],
                          cv['Wk'][:, EMB:], cv['Wv'][:, EMB:],
                          cv['Wq'], cv['Wskip']], axis=1)
  ball = jnp.concatenate([cv['bk'][:EMB], cv['bv'][:EMB],
                          cv['bk'][EMB:], cv['bv'][EMB:],
                          cv['bq'], cv['bskip']])[None, :]
  wbeta = cv['Wbeta'][:, 0]
  w1, w2, w3 = wbeta[:HC], wbeta[HC:2 * HC], wbeta[2 * HC:]
  wa = (w1 + w3)[None, :]
  wb = (w2 - w3)[None, :]
  return {
      'wall': wall, 'ball': ball,
      'we': cv['We'], 'be': cv['be'][None, :],
      'wa': wa, 'wb': wb,
      'wt': p['Wt'], 'bt': p['bt'][None, :],
      'g': p['bn_g'][None, :], 'b': p['bn_b'][None, :],
  }


def kernel(x, edge_attr, edge_index, batch_index, params):
  f32 = jnp.float32
  src2 = jnp.pad(edge_index[0], (0, E_PAD - N_EDGES)).reshape(N_CHROWS, CH)
  dst2 = jnp.pad(edge_index[1], (0, E_PAD - N_EDGES)).reshape(N_CHROWS, CH)
  src3 = jnp.stack([src2, src2 + N_NODES], axis=0)
  dst3 = jnp.stack([dst2, dst2 + N_NODES], axis=0)
  ea_pad = jnp.pad(edge_attr, ((0, E_PAD - N_EDGES), (0, 0)))
  bif = batch_index.astype(f32).reshape(N_NODES, 1)

  zrows = jnp.zeros((ROWS_PER_TILE, HC), f32)
  zstats = jnp.zeros((2, D_FEAT), f32)
  zvec = jnp.zeros((1, D_FEAT), f32)

  blocks = [_prep_block(params['block0'])] + [
      _prep_block(p) for p in params['layers']]

  rep = jnp.zeros((N_GRAPHS, 2 * EMB), f32)
  h2, stats = None, None
  for li, bp in enumerate(blocks):
    if li == 0:
      t, q, sk = _run_matmul(x, bp['wall'], bp['ball'],
                             zstats, zvec, zvec, norm=False)
    else:
      t, q, sk = _run_matmul(h2, bp['wall'], bp['ball'],
                             stats, blocks[li - 1]['g'], blocks[li - 1]['b'],
                             norm=True)
    eh = _run_eproj(ea_pad, bp['we'], bp['be'])
    tf = t.reshape(NC * N_NODES, HC)
    qf = q.reshape(NC * N_NODES, HC)
    part = _edge_sc(tf, qf, eh, src3, dst3, dst2, zrows)
    h2, stats = _run_combine(part, sk, bp['wa'], bp['wb'],
                             bp['wt'], bp['bt'])
    if li >= 1:
      rep = _run_pool(h2, stats, bp['g'], bp['b'], bif, rep)
  return rep


# speedup vs baseline: 1.1736x; 1.1736x over previous
import functools

import jax
import jax.numpy as jnp
from jax import lax
from jax.experimental import pallas as pl
from jax.experimental.pallas import tpu as pltpu
from jax.experimental.pallas import tpu_sc as plsc

N_NODES = 10000
N_EDGES = 160000
D_FEAT = 128
D_EDGE = 16
EMB = 64
HEADS = 2
N_GRAPHS = 32
HC = HEADS * EMB

NC = 2
NS = 16
E_PAD = 163840
CH = 64
N_CHROWS = E_PAD // CH
CPT = N_CHROWS // NS
IGRP = 16
N_IGRP = CPT // IGRP
N_ACC = 10240
ROWS_PER_TILE = N_ACC // NS

EB = 1280
N_EBLK = E_PAD // EB
NB = 1000
N_NBLK = N_NODES // NB
PB = 400
N_PBLK = N_NODES // PB

_mesh = plsc.VectorSubcoreMesh(
    core_axis_name="c", subcore_axis_name="s", num_cores=NC, num_subcores=NS)

_GDN = lax.GatherDimensionNumbers(
    offset_dims=(), collapsed_slice_dims=(0,), start_index_map=(0,))


def _lanesum(v):
  for k in (1, 2, 4, 8):
    perm = lax.iota(jnp.int32, 16) ^ k
    v = v + lax.gather(v, perm[:, None], _GDN, (1,),
                       mode=lax.GatherScatterMode.PROMISE_IN_BOUNDS)
  return v


def _edge_sc_body(t_hbm, q_hbm, e_hbm, src_hbm, dstq_hbm, dst_hbm, zrows_hbm,
                  part_hbm, acc, idx_s, idx_dq, idx_d, tbuf, qbuf, ebuf, obuf,
                  sem_t, sem_q, sem_e):
  cid = lax.axis_index("c")
  sid = lax.axis_index("s")
  pltpu.sync_copy(zrows_hbm, acc.at[pl.ds(sid * ROWS_PER_TILE, ROWS_PER_TILE)])

  def zrow(r, _):
    z16 = jnp.zeros((16,), jnp.float32)
    obuf[r, 80:96] = z16
    obuf[r, 96:112] = z16
    obuf[r, 112:128] = z16
    return 0

  lax.fori_loop(0, CH, zrow, 0)
  plsc.subcore_barrier()

  row0 = sid * CPT

  def group(g, _):
    grow = row0 + g * IGRP
    pltpu.sync_copy(src_hbm.at[cid, pl.ds(grow, IGRP)], idx_s)
    pltpu.sync_copy(dstq_hbm.at[cid, pl.ds(grow, IGRP)], idx_dq)
    pltpu.sync_copy(dst_hbm.at[pl.ds(grow, IGRP)], idx_d)

    def chunk(j2, _):
      base = (grow + j2) * CH
      ct = pltpu.make_async_copy(t_hbm.at[idx_s.at[j2]], tbuf, sem_t)
      cq = pltpu.make_async_copy(q_hbm.at[idx_dq.at[j2]], qbuf, sem_q)
      ce = pltpu.make_async_copy(e_hbm.at[cid, pl.ds(base, CH)], ebuf, sem_e)
      ct.start()
      cq.start()
      ce.start()
      ct.wait()
      cq.wait()
      ce.wait()

      def edge(r, _):
        e0 = ebuf[r, 0:16]
        e1 = ebuf[r, 16:32]
        e2 = ebuf[r, 32:48]
        e3 = ebuf[r, 48:64]
        s16 = (qbuf[r, 0:16] * (tbuf[r, 0:16] + e0)
               + qbuf[r, 16:32] * (tbuf[r, 16:32] + e1)
               + qbuf[r, 32:48] * (tbuf[r, 32:48] + e2)
               + qbuf[r, 48:64] * (tbuf[r, 48:64] + e3))
        live = (base + r < N_EDGES).astype(jnp.float32)
        exv = jnp.exp(_lanesum(s16) * 0.125) * live
        obuf[r, 0:16] = (tbuf[r, 64:80] + e0) * exv
        obuf[r, 16:32] = (tbuf[r, 80:96] + e1) * exv
        obuf[r, 32:48] = (tbuf[r, 96:112] + e2) * exv
        obuf[r, 48:64] = (tbuf[r, 112:128] + e3) * exv
        obuf[r, 64:80] = exv
      pltpu.sync_copy(obuf, acc.at[idx_d.at[j2]], add=True)
      return 0

    lax.fori_loop(0, IGRP, chunk, 0)
    return 0

  lax.fori_loop(0, N_IGRP, group, 0)
  plsc.subcore_barrier()
  pltpu.sync_copy(acc.at[pl.ds(sid * ROWS_PER_TILE, ROWS_PER_TILE)],
                  part_hbm.at[cid, pl.ds(sid * ROWS_PER_TILE, ROWS_PER_TILE)])


_edge_sc = pl.kernel(
    _edge_sc_body,
    out_type=jax.ShapeDtypeStruct((NC, N_ACC, HC), jnp.float32),
    mesh=_mesh,
    scratch_types=[pltpu.VMEM_SHARED((N_ACC, HC), jnp.float32),
                   pltpu.VMEM((IGRP, CH), jnp.int32),
                   pltpu.VMEM((IGRP, CH), jnp.int32),
                   pltpu.VMEM((IGRP, CH), jnp.int32),
                   pltpu.VMEM((CH, HC), jnp.float32),
                   pltpu.VMEM((CH, HC), jnp.float32),
                   pltpu.VMEM((CH, EMB), jnp.float32),
                   pltpu.VMEM((CH, HC), jnp.float32),
                   pltpu.SemaphoreType.DMA,
                   pltpu.SemaphoreType.DMA,
                   pltpu.SemaphoreType.DMA],
)


def _mm_body(norm, x_ref, w_ref, b_ref, st_ref, g_ref, bb_ref,
             t_ref, q_ref, sk_ref):
  x = x_ref[...]
  if norm:
    inv_n = 1.0 / N_NODES
    mu = st_ref[0:1, :] * inv_n
    var = st_ref[1:2, :] * inv_n - mu * mu
    s = g_ref[...] * lax.rsqrt(var + 1e-5)
    t = bb_ref[...] - mu * s
    x = x * s + t
  y = jnp.dot(x, w_ref[...], preferred_element_type=jnp.float32) + b_ref[...]
  t_ref[...] = jnp.stack([y[:, :HC], y[:, HC:2 * HC]], axis=0)
  z = jnp.zeros((y.shape[0], EMB), jnp.float32)
  q_ref[...] = jnp.stack(
      [jnp.concatenate([y[:, 2 * HC:2 * HC + EMB], z], axis=1),
       jnp.concatenate([y[:, 2 * HC + EMB:3 * HC], z], axis=1)], axis=0)
  sk_ref[...] = y[:, 3 * HC:]


def _run_matmul(x, wall, ball, stats, g, b, norm):
  d = x.shape[1]
  body = functools.partial(_mm_body, norm)
  return pl.pallas_call(
      body,
      grid=(N_NBLK,),
      in_specs=[
          pl.BlockSpec((NB, d), lambda i: (i, 0)),
          pl.BlockSpec((d, 4 * HC), lambda i: (0, 0)),
          pl.BlockSpec((1, 4 * HC), lambda i: (0, 0)),
          pl.BlockSpec((2, d), lambda i: (0, 0)),
          pl.BlockSpec((1, d), lambda i: (0, 0)),
          pl.BlockSpec((1, d), lambda i: (0, 0)),
      ],
      out_specs=[
          pl.BlockSpec((NC, NB, HC), lambda i: (0, i, 0)),
          pl.BlockSpec((NC, NB, HC), lambda i: (0, i, 0)),
          pl.BlockSpec((NB, HC), lambda i: (i, 0)),
      ],
      out_shape=[
          jax.ShapeDtypeStruct((NC, N_NODES, HC), jnp.float32),
          jax.ShapeDtypeStruct((NC, N_NODES, HC), jnp.float32),
          jax.ShapeDtypeStruct((N_NODES, HC), jnp.float32),
      ],
  )(x, wall, ball, stats, g, b)


def _eproj_body(ea_ref, we_ref, be_ref, eh_ref):
  e = jnp.dot(ea_ref[...], we_ref[...],
              preferred_element_type=jnp.float32) + be_ref[...]
  eh_ref[...] = jnp.stack([e[:, :EMB], e[:, EMB:]], axis=0)


def _run_eproj(ea_pad, we, be):
  return pl.pallas_call(
      _eproj_body,
      grid=(N_EBLK,),
      in_specs=[
          pl.BlockSpec((EB, D_EDGE), lambda i: (i, 0)),
          pl.BlockSpec((D_EDGE, HC), lambda i: (0, 0)),
          pl.BlockSpec((1, HC), lambda i: (0, 0)),
      ],
      out_specs=pl.BlockSpec((NC, EB, EMB), lambda i: (0, i, 0)),
      out_shape=jax.ShapeDtypeStruct((NC, E_PAD, EMB), jnp.float32),
  )(ea_pad, we, be)


def _combine_body(part_ref, sk_ref, wa_ref, wb_ref,
                  wt_ref, bt_ref, h2_ref, st_ref, sacc):
  pid = pl.program_id(0)
  p = part_ref[...]
  num = jnp.concatenate([p[0][:, :EMB], p[1][:, :EMB]], axis=1)
  d0 = p[0][:, EMB:EMB + 16]
  d1 = p[1][:, EMB:EMB + 16]
  den = jnp.concatenate([d0, d0, d0, d0, d1, d1, d1, d1], axis=1)
  out = num / (den + 1e-16)
  xr = sk_ref[...]
  bpre = (jnp.sum(out * wa_ref[...], axis=1, keepdims=True)
          + jnp.sum(xr * wb_ref[...], axis=1, keepdims=True))
  beta = jax.nn.sigmoid(bpre)
  h = beta * xr + (1.0 - beta) * out
  y = jnp.dot(h, wt_ref[...], preferred_element_type=jnp.float32) + bt_ref[...]
  h2 = jnp.where(y > 0, y, 0.01 * y)
  h2_ref[...] = h2

  @pl.when(pid == 0)
  def _():
    sacc[...] = jnp.zeros_like(sacc)

  sacc[0:1, :] += jnp.sum(h2, axis=0, keepdims=True)
  sacc[1:2, :] += jnp.sum(h2 * h2, axis=0, keepdims=True)

  @pl.when(pid == N_NBLK - 1)
  def _():
    st_ref[...] = sacc[...]


def _run_combine(part, sk, wa, wb, wt, bt):
  return pl.pallas_call(
      _combine_body,
      grid=(N_NBLK,),
      in_specs=[
          pl.BlockSpec((NC, NB, HC), lambda i: (0, i, 0)),
          pl.BlockSpec((NB, HC), lambda i: (i, 0)),
          pl.BlockSpec((1, HC), lambda i: (0, 0)),
          pl.BlockSpec((1, HC), lambda i: (0, 0)),
          pl.BlockSpec((HC, EMB), lambda i: (0, 0)),
          pl.BlockSpec((1, EMB), lambda i: (0, 0)),
      ],
      out_specs=[
          pl.BlockSpec((NB, EMB), lambda i: (i, 0)),
          pl.BlockSpec((2, EMB), lambda i: (0, 0)),
      ],
      out_shape=[
          jax.ShapeDtypeStruct((N_NODES, EMB), jnp.float32),
          jax.ShapeDtypeStruct((2, EMB), jnp.float32),
      ],
      scratch_shapes=[pltpu.VMEM((2, EMB), jnp.float32)],
  )(part, sk, wa, wb, wt, bt)


def _pool_body(h2_ref, st_ref, g_ref, bb_ref, bi_ref, prev_ref, rep_ref,
               mxacc, smacc, ctacc):
  pid = pl.program_id(0)
  inv_n = 1.0 / N_NODES
  mu = st_ref[0:1, :] * inv_n
  var = st_ref[1:2, :] * inv_n - mu * mu
  s = g_ref[...] * lax.rsqrt(var + 1e-5)
  t = bb_ref[...] - mu * s
  xn = h2_ref[...] * s + t
  bif = bi_ref[...]
  neg = jnp.float32(-jnp.inf)

  @pl.when(pid == 0)
  def _():
    mxacc[...] = jnp.full_like(mxacc, neg)
    smacc[...] = jnp.zeros_like(smacc)
    ctacc[...] = jnp.zeros_like(ctacc)

  for g in range(N_GRAPHS):
    mk = bif == float(g)
    sel = jnp.where(mk, xn, neg)
    mxacc[g:g + 1, :] = jnp.maximum(
        mxacc[g:g + 1, :], jnp.max(sel, axis=0, keepdims=True))
    smacc[g:g + 1, :] += jnp.sum(jnp.where(mk, xn, 0.0), axis=0, keepdims=True)
    ctacc[g:g + 1, :] += jnp.sum(mk.astype(jnp.float32), axis=0, keepdims=True)

  @pl.when(pid == N_PBLK - 1)
  def _():
    mean = smacc[...] / jnp.maximum(ctacc[...], 1.0)
    mx = mxacc[...]
    mx = jnp.where(mx == neg, 0.0, mx)
    rep_ref[...] = jnp.concatenate([mx, mean], axis=1) + prev_ref[...]


def _run_pool(h2, stats, g, b, bif, prev):
  return pl.pallas_call(
      _pool_body,
      grid=(N_PBLK,),
      in_specs=[
          pl.BlockSpec((PB, EMB), lambda i: (i, 0)),
          pl.BlockSpec((2, EMB), lambda i: (0, 0)),
          pl.BlockSpec((1, EMB), lambda i: (0, 0)),
          pl.BlockSpec((1, EMB), lambda i: (0, 0)),
          pl.BlockSpec((PB, 1), lambda i: (i, 0)),
          pl.BlockSpec((N_GRAPHS, 2 * EMB), lambda i: (0, 0)),
      ],
      out_specs=pl.BlockSpec((N_GRAPHS, 2 * EMB), lambda i: (0, 0)),
      out_shape=jax.ShapeDtypeStruct((N_GRAPHS, 2 * EMB), jnp.float32),
      scratch_shapes=[pltpu.VMEM((N_GRAPHS, EMB), jnp.float32),
                      pltpu.VMEM((N_GRAPHS, EMB), jnp.float32),
                      pltpu.VMEM((N_GRAPHS, 1), jnp.float32)],
  )(h2, stats, g, b, bif, prev)


def _prep_block(p):
  cv = p['conv']
  wall = jnp.concatenate([cv['Wk'][:, :EMB], cv['Wv'][:, :EMB],
                          cv['Wk'][:, EMB:], cv['Wv'][:, EMB:],
                          cv['Wq'], cv['Wskip']], axis=1)
  ball = jnp.concatenate([cv['bk'][:EMB], cv['bv'][:EMB],
                          cv['bk'][EMB:], cv['bv'][EMB:],
                          cv['bq'], cv['bskip']])[None, :]
  wbeta = cv['Wbeta'][:, 0]
  w1, w2, w3 = wbeta[:HC], wbeta[HC:2 * HC], wbeta[2 * HC:]
  wa = (w1 + w3)[None, :]
  wb = (w2 - w3)[None, :]
  return {
      'wall': wall, 'ball': ball,
      'we': cv['We'], 'be': cv['be'][None, :],
      'wa': wa, 'wb': wb,
      'wt': p['Wt'], 'bt': p['bt'][None, :],
      'g': p['bn_g'][None, :], 'b': p['bn_b'][None, :],
  }


def kernel(x, edge_attr, edge_index, batch_index, params):
  f32 = jnp.float32
  src2 = jnp.pad(edge_index[0], (0, E_PAD - N_EDGES)).reshape(N_CHROWS, CH)
  dst2 = jnp.pad(edge_index[1], (0, E_PAD - N_EDGES)).reshape(N_CHROWS, CH)
  src3 = jnp.stack([src2, src2 + N_NODES], axis=0)
  dst3 = jnp.stack([dst2, dst2 + N_NODES], axis=0)
  ea_pad = jnp.pad(edge_attr, ((0, E_PAD - N_EDGES), (0, 0)))
  bif = batch_index.astype(f32).reshape(N_NODES, 1)

  zrows = jnp.zeros((ROWS_PER_TILE, HC), f32)
  zstats = jnp.zeros((2, D_FEAT), f32)
  zvec = jnp.zeros((1, D_FEAT), f32)

  blocks = [_prep_block(params['block0'])] + [
      _prep_block(p) for p in params['layers']]

  rep = jnp.zeros((N_GRAPHS, 2 * EMB), f32)
  h2, stats = None, None
  for li, bp in enumerate(blocks):
    if li == 0:
      t, q, sk = _run_matmul(x, bp['wall'], bp['ball'],
                             zstats, zvec, zvec, norm=False)
    else:
      t, q, sk = _run_matmul(h2, bp['wall'], bp['ball'],
                             stats, blocks[li - 1]['g'], blocks[li - 1]['b'],
                             norm=True)
    eh = _run_eproj(ea_pad, bp['we'], bp['be'])
    tf = t.reshape(NC * N_NODES, HC)
    qf = q.reshape(NC * N_NODES, HC)
    part = _edge_sc(tf, qf, eh, src3, dst3, dst2, zrows)
    h2, stats = _run_combine(part, sk, bp['wa'], bp['wb'],
                             bp['wt'], bp['bt'])
    if li >= 1:
      rep = _run_pool(h2, stats, bp['g'], bp['b'], bif, rep)
  return rep
